# Initial kernel scaffold; baseline (speedup 1.0000x reference)
import jax
import jax.numpy as jnp
from jax.experimental import pallas as pl


def kernel(x):
    raise NotImplementedError("write your pallas kernel here")


# speedup vs baseline: 2.6618x; 2.6618x over previous
import functools

import jax
import jax.numpy as jnp
from jax.experimental import pallas as pl

K = 64
ROWS_PER_BLOCK = 8
NEG_INF = float("-inf")


def _topk_body(x_ref, o_ref):
    x = x_ref[...]
    rb, n = x.shape
    iota = jax.lax.broadcasted_iota(jnp.int32, (rb, n), 1)
    iota_k = jax.lax.broadcasted_iota(jnp.int32, (rb, K), 1)

    def step(j, carry):
        cur, outs = carry
        m = jnp.max(cur, axis=1, keepdims=True)
        outs = jnp.where(iota_k == j, m, outs)
        pos = jnp.min(jnp.where(cur == m, iota, n), axis=1, keepdims=True)
        return jnp.where(iota == pos, NEG_INF, cur), outs

    _, outs = jax.lax.fori_loop(
        0, K, step, (x, jnp.zeros((rb, K), dtype=x.dtype))
    )
    o_ref[...] = outs


def kernel(x):
    B0, B1, N = x.shape
    x2 = x.reshape(B0 * B1, N)
    R = B0 * B1
    out = pl.pallas_call(
        _topk_body,
        grid=(R // ROWS_PER_BLOCK,),
        in_specs=[pl.BlockSpec((ROWS_PER_BLOCK, N), lambda i: (i, 0))],
        out_specs=pl.BlockSpec((ROWS_PER_BLOCK, K), lambda i: (i, 0)),
        out_shape=jax.ShapeDtypeStruct((R, K), x.dtype),
    )(x2)
    return out.reshape(B0, B1, K)


# speedup vs baseline: 9.3031x; 3.4950x over previous
import jax
import jax.numpy as jnp
from jax.experimental import pallas as pl

K = 64
N = 32768
ROWS_PER_BLOCK = 8
_HI = jax.lax.Precision.HIGHEST


def _col_ranks(a):
    b, g = a.shape
    ap = a[:, :, None]
    aq = a[:, None, :]
    ip = jax.lax.broadcasted_iota(jnp.int32, (b, g, g), 1)
    iq = jax.lax.broadcasted_iota(jnp.int32, (b, g, g), 2)
    beats = (aq > ap) | ((aq == ap) & (iq < ip))
    return jnp.sum(beats.astype(jnp.int32), axis=2)


def _level(y):
    b, s, g = y.shape
    a = jnp.max(y, axis=1)
    rank = _col_ranks(a)
    im = jax.lax.broadcasted_iota(jnp.int32, (b, g, K), 2)
    p = (rank[:, :, None] == im).astype(jnp.float32)
    yc = jax.lax.dot_general(
        y, p, (((2,), (1,)), ((0,), (0,))),
        precision=_HI, preferred_element_type=jnp.float32)
    s2 = s // 2
    return jnp.concatenate([yc[:, :s2], yc[:, s2:]], axis=2)


def _topk_body(x_ref, o_ref):
    y = x_ref[...]
    b = y.shape[0]
    for _ in range(8):
        y = _level(y)
    v = y.reshape(b, 128)
    rank = _col_ranks(v)
    im = jax.lax.broadcasted_iota(jnp.int32, (b, K, 128), 1)
    sel = (rank[:, None, :] == im).astype(jnp.float32)
    o_ref[...] = jnp.sum(sel * v[:, None, :], axis=2)


def kernel(x):
    b0, b1, n = x.shape
    r = b0 * b1
    x3 = x.reshape(r, n // 128, 128)
    out = pl.pallas_call(
        _topk_body,
        grid=(r // ROWS_PER_BLOCK,),
        in_specs=[pl.BlockSpec((ROWS_PER_BLOCK, n // 128, 128),
                               lambda i: (i, 0, 0))],
        out_specs=pl.BlockSpec((ROWS_PER_BLOCK, K), lambda i: (i, 0)),
        out_shape=jax.ShapeDtypeStruct((r, K), x.dtype),
    )(x3)
    return out.reshape(b0, b1, K)


# speedup vs baseline: 12.9803x; 1.3953x over previous
import jax
import jax.numpy as jnp
from jax.experimental import pallas as pl
from jax.experimental.pallas import tpu as pltpu

K = 64
ROWS_PER_BLOCK = 8
_HI = jax.lax.Precision.HIGHEST


def _sort128_desc(v):
    b, n = v.shape
    j = jax.lax.broadcasted_iota(jnp.int32, (b, n), 1)
    for k in (2, 4, 8, 16, 32, 64, 128):
        d = k // 2
        while d >= 1:
            lower = (j & d) == 0
            partner = jnp.where(lower,
                                pltpu.roll(v, n - d, 1),
                                pltpu.roll(v, d, 1))
            keep_max = lower ^ ((j & k) != 0)
            v = jnp.where(keep_max,
                          jnp.maximum(v, partner),
                          jnp.minimum(v, partner))
            d //= 2
    return v


def _col_ranks(a):
    b, g = a.shape
    ap = a[:, :, None]
    aq = a[:, None, :]
    ip = jax.lax.broadcasted_iota(jnp.int32, (b, g, g), 1)
    iq = jax.lax.broadcasted_iota(jnp.int32, (b, g, g), 2)
    beats = (aq > ap) | ((aq == ap) & (iq < ip))
    return jnp.sum(beats.astype(jnp.int32), axis=2)


def _level(y):
    b, s, g = y.shape
    a = jnp.max(y, axis=1)
    rank = _col_ranks(a)
    im = jax.lax.broadcasted_iota(jnp.int32, (b, g, K), 2)
    p = (rank[:, :, None] == im).astype(jnp.float32)
    yc = jax.lax.dot_general(
        y, p, (((2,), (1,)), ((0,), (0,))),
        precision=_HI, preferred_element_type=jnp.float32)
    s2 = s // 2
    return jnp.concatenate([yc[:, :s2], yc[:, s2:]], axis=2)


def _topk_body(x_ref, o_ref):
    y = x_ref[...]
    b = y.shape[0]
    for _ in range(8):
        y = _level(y)
    v = _sort128_desc(y.reshape(b, 128))
    o_ref[...] = jax.lax.slice(v, (0, 0), (b, K))


def kernel(x):
    b0, b1, n = x.shape
    r = b0 * b1
    x3 = x.reshape(r, n // 128, 128)
    out = pl.pallas_call(
        _topk_body,
        grid=(r // ROWS_PER_BLOCK,),
        in_specs=[pl.BlockSpec((ROWS_PER_BLOCK, n // 128, 128),
                               lambda i: (i, 0, 0))],
        out_specs=pl.BlockSpec((ROWS_PER_BLOCK, K), lambda i: (i, 0)),
        out_shape=jax.ShapeDtypeStruct((r, K), x.dtype),
    )(x3)
    return out.reshape(b0, b1, K)


# speedup vs baseline: 15.9297x; 1.2272x over previous
import jax
import jax.numpy as jnp
from jax.experimental import pallas as pl
from jax.experimental.pallas import tpu as pltpu

K = 64
ROWS_PER_BLOCK = 8
_HI = jax.lax.Precision.HIGHEST


def _sort128_desc(v):
    b, n = v.shape
    j = jax.lax.broadcasted_iota(jnp.int32, (b, n), 1)
    for k in (2, 4, 8, 16, 32, 64, 128):
        d = k // 2
        while d >= 1:
            lower = (j & d) == 0
            partner = jnp.where(lower,
                                pltpu.roll(v, n - d, 1),
                                pltpu.roll(v, d, 1))
            keep_max = lower ^ ((j & k) != 0)
            v = jnp.where(keep_max,
                          jnp.maximum(v, partner),
                          jnp.minimum(v, partner))
            d //= 2
    return v


def _level(y):
    b, s, g = y.shape
    a = jnp.max(y, axis=1)
    ap = a[:, :, None]
    aq = a[:, None, :]
    ip = jax.lax.broadcasted_iota(jnp.int32, (b, g, g), 1)
    iq = jax.lax.broadcasted_iota(jnp.int32, (b, g, g), 2)
    beats = ((aq > ap) | ((aq == ap) & (iq < ip))).astype(jnp.float32)
    ones = jnp.ones((b, g, 1), dtype=jnp.float32)
    rank = jax.lax.dot_general(
        beats, ones, (((2,), (1,)), ((0,), (0,))),
        preferred_element_type=jnp.float32)
    im = jax.lax.broadcasted_iota(jnp.int32, (b, g, K), 2)
    p = (rank.astype(jnp.int32) == im).astype(jnp.float32)
    yc = jax.lax.dot_general(
        y, p, (((2,), (1,)), ((0,), (0,))),
        precision=_HI, preferred_element_type=jnp.float32)
    s2 = s // 2
    return jnp.concatenate([yc[:, :s2], yc[:, s2:]], axis=2)


def _topk_body(x_ref, o_ref):
    y = x_ref[...]
    b = y.shape[0]
    for _ in range(8):
        y = _level(y)
    v = _sort128_desc(y.reshape(b, 128))
    o_ref[...] = jax.lax.slice(v, (0, 0), (b, K))


def kernel(x):
    b0, b1, n = x.shape
    r = b0 * b1
    x3 = x.reshape(r, n // 128, 128)
    out = pl.pallas_call(
        _topk_body,
        grid=(r // ROWS_PER_BLOCK,),
        in_specs=[pl.BlockSpec((ROWS_PER_BLOCK, n // 128, 128),
                               lambda i: (i, 0, 0))],
        out_specs=pl.BlockSpec((ROWS_PER_BLOCK, K), lambda i: (i, 0)),
        out_shape=jax.ShapeDtypeStruct((r, K), x.dtype),
    )(x3)
    return out.reshape(b0, b1, K)
